# baseline (speedup 1.0000x reference)
import functools

import jax
import jax.numpy as jnp
from jax import lax
from jax.experimental import pallas as pl
from jax.experimental.pallas import tpu as pltpu
from jax.experimental.pallas import tpu_sc as plsc

N = 10000
D = 128
NPAD = 10240
NC = 2
NS = 16
NW = NC * NS
E2 = 640000
EPT = E2 // NW
BD = 80
CHUNKS = EPT // BD
B = 128
SBC = 32
SB = 5
NB = SB * SBC
EPTP = NB * B
RPT = NPAD // NS
ZR = 16
BLK = 1024
TRASH = NPAD - 2


def _mesh():
    return plsc.VectorSubcoreMesh(
        core_axis_name="c", subcore_axis_name="s", num_cores=NC, num_subcores=NS
    )


@functools.partial(
    pl.kernel,
    out_type=jax.ShapeDtypeStruct((NC, NPAD), jnp.float32),
    mesh=_mesh(),
    scratch_types=[
        pltpu.VMEM((CHUNKS, BD), jnp.int32),
        pltpu.VMEM((BD,), jnp.float32),
        pltpu.VMEM((RPT,), jnp.float32),
        pltpu.VMEM_SHARED((NPAD,), jnp.float32),
    ],
)
def _deg_call(col_hbm, z1_hbm, out_hbm, idx_v, ones_v, rbuf_v, deg_sh):
    c = lax.axis_index("c")
    s = lax.axis_index("s")
    wid = c * NS + s
    pltpu.sync_copy(z1_hbm, rbuf_v)
    pltpu.sync_copy(rbuf_v, deg_sh.at[pl.ds(s * RPT, RPT)])
    for i in range(BD // 16):
        ones_v[pl.ds(i * 16, 16)] = jnp.full((16,), 1.0, jnp.float32)
    pltpu.sync_copy(col_hbm.at[wid], idx_v)
    plsc.subcore_barrier()

    @pl.loop(0, CHUNKS)
    def _(j):
        pltpu.sync_copy(ones_v, deg_sh.at[idx_v.at[j]], add=True)

    plsc.subcore_barrier()
    pltpu.sync_copy(deg_sh.at[pl.ds(s * RPT, RPT)], rbuf_v)
    pltpu.sync_copy(rbuf_v, out_hbm.at[c, pl.ds(s * RPT, RPT)])


@functools.partial(
    pl.kernel,
    out_type=jax.ShapeDtypeStruct((NC, NPAD, D), jnp.float32),
    mesh=_mesh(),
    scratch_types=[
        pltpu.VMEM((SBC, B), jnp.int32),
        pltpu.VMEM((SBC, B), jnp.int32),
        pltpu.VMEM((B, D), jnp.float32),
        pltpu.VMEM((B, D), jnp.float32),
        pltpu.VMEM((ZR, D), jnp.float32),
        pltpu.VMEM_SHARED((NPAD, D), jnp.float32),
        pltpu.SemaphoreType.DMA,
        pltpu.SemaphoreType.DMA,
    ],
)
def _prop_call(t_hbm, row_hbm, col_hbm, z2_hbm, out_hbm,
               ridx_v, cidx_v, rows_a, rows_b, zbuf_v, acc_sh, sem_a, sem_b):
    c = lax.axis_index("c")
    s = lax.axis_index("s")
    wid = c * NS + s

    def gather(j, buf, sem):
        return pltpu.async_copy(t_hbm.at[ridx_v.at[j]], buf, sem)

    def gwait(j, buf, sem):
        pltpu.make_async_copy(t_hbm.at[ridx_v.at[j]], buf, sem).wait()

    def scat(j, buf):
        pltpu.sync_copy(buf, acc_sh.at[cidx_v.at[j]], add=True)
    pltpu.sync_copy(z2_hbm, zbuf_v)
    for r in range(RPT // ZR):
        pltpu.sync_copy(zbuf_v, acc_sh.at[pl.ds(s * RPT + r * ZR, ZR)])
    plsc.subcore_barrier()

    @pl.loop(0, SB)
    def _(q):
        pltpu.sync_copy(row_hbm.at[wid, q], ridx_v)
        pltpu.sync_copy(col_hbm.at[wid, q], cidx_v)
        gather(0, rows_a, sem_a)

        @pl.loop(0, SBC // 2 - 1)
        def _(p):
            j = p * 2
            gather(j + 1, rows_b, sem_b)
            gwait(j, rows_a, sem_a)
            scat(j, rows_a)
            gather(j + 2, rows_a, sem_a)
            gwait(j + 1, rows_b, sem_b)
            scat(j + 1, rows_b)

        gather(SBC - 1, rows_b, sem_b)
        gwait(SBC - 2, rows_a, sem_a)
        scat(SBC - 2, rows_a)
        gwait(SBC - 1, rows_b, sem_b)
        scat(SBC - 1, rows_b)

    plsc.subcore_barrier()
    for r in range(RPT // ZR):
        base = s * RPT + r * ZR
        pltpu.sync_copy(acc_sh.at[pl.ds(base, ZR)], zbuf_v)
        pltpu.sync_copy(zbuf_v, out_hbm.at[c, pl.ds(base, ZR)])


def _prep_body(degp_ref, x_ref, t0_ref, dinv_ref):
    deg = degp_ref[0] + degp_ref[1]
    di = jnp.where(deg > 0.0, lax.rsqrt(deg), 0.0)
    dinv_ref[...] = di
    t0_ref[...] = di * x_ref[...]


_prep_call = pl.pallas_call(
    _prep_body,
    grid=(NPAD // BLK,),
    in_specs=[
        pl.BlockSpec((2, BLK, 1), lambda i: (0, i, 0)),
        pl.BlockSpec((BLK, D), lambda i: (i, 0)),
    ],
    out_specs=[
        pl.BlockSpec((BLK, D), lambda i: (i, 0)),
        pl.BlockSpec((BLK, 1), lambda i: (i, 0)),
    ],
    out_shape=[
        jax.ShapeDtypeStruct((NPAD, D), jnp.float32),
        jax.ShapeDtypeStruct((NPAD, 1), jnp.float32),
    ],
)


def _comb_body(p_ref, dinv_ref, r_ref, t_ref, rout_ref):
    ssum = p_ref[0] + p_ref[1]
    di = dinv_ref[...]
    contrib = di * ssum
    t_ref[...] = di * contrib
    rout_ref[...] = r_ref[...] + contrib


_comb_call = pl.pallas_call(
    _comb_body,
    grid=(NPAD // BLK,),
    in_specs=[
        pl.BlockSpec((2, BLK, D), lambda i: (0, i, 0)),
        pl.BlockSpec((BLK, 1), lambda i: (i, 0)),
        pl.BlockSpec((BLK, D), lambda i: (i, 0)),
    ],
    out_specs=[
        pl.BlockSpec((BLK, D), lambda i: (i, 0)),
        pl.BlockSpec((BLK, D), lambda i: (i, 0)),
    ],
    out_shape=[
        jax.ShapeDtypeStruct((NPAD, D), jnp.float32),
        jax.ShapeDtypeStruct((NPAD, D), jnp.float32),
    ],
)


def _fin_body(p_ref, dinv_ref, r_ref, o_ref):
    ssum = p_ref[0] + p_ref[1]
    o_ref[...] = (r_ref[...] + dinv_ref[...] * ssum) * 0.25


_fin_call = pl.pallas_call(
    _fin_body,
    grid=(NPAD // BLK,),
    in_specs=[
        pl.BlockSpec((2, BLK, D), lambda i: (0, i, 0)),
        pl.BlockSpec((BLK, 1), lambda i: (i, 0)),
        pl.BlockSpec((BLK, D), lambda i: (i, 0)),
    ],
    out_specs=pl.BlockSpec((BLK, D), lambda i: (i, 0)),
    out_shape=jax.ShapeDtypeStruct((NPAD, D), jnp.float32),
)


def kernel(x, adj_t):
    a0 = adj_t[0].astype(jnp.int32)
    a1 = adj_t[1].astype(jnp.int32)
    row2 = jnp.concatenate([a0, a1]).reshape(NW, EPT)
    col2 = jnp.concatenate([a1, a0]).reshape(NW, EPT)
    col_deg = col2.reshape(NW, CHUNKS, BD)
    pad_r = jnp.zeros((NW, EPTP - EPT), jnp.int32)
    pad_c = jnp.full((NW, EPTP - EPT), TRASH, jnp.int32)
    row = jnp.concatenate([row2, pad_r], axis=1).reshape(NW, SB, SBC, B)
    col = jnp.concatenate([col2, pad_c], axis=1).reshape(NW, SB, SBC, B)
    xpad = jnp.concatenate([x, jnp.zeros((NPAD - N, D), jnp.float32)], axis=0)
    z1 = jnp.zeros((RPT,), jnp.float32)
    z2 = jnp.zeros((ZR, D), jnp.float32)

    degp = _deg_call(col_deg, z1)
    t, dinv = _prep_call(degp.reshape(NC, NPAD, 1), xpad)
    r = xpad
    for layer in range(3):
        p = _prop_call(t, row, col, z2)
        if layer < 2:
            t, r = _comb_call(p, dinv, r)
        else:
            out = _fin_call(p, dinv, r)
    return out[:N]


# speedup vs baseline: 1.9456x; 1.9456x over previous
import functools

import jax
import jax.numpy as jnp
from jax import lax
from jax.experimental import pallas as pl
from jax.experimental.pallas import tpu as pltpu
from jax.experimental.pallas import tpu_sc as plsc

N = 10000
D = 128
NPAD = 10240
NC = 2
NS = 16
NW = NC * NS
E2 = 640000
EPT = E2 // NW
BD = 80
CHUNKS = EPT // BD
B = 80
SBC = 63
SB = 4
NB = SB * SBC
EPTP = NB * B
RPT = NPAD // NS
ZR = 16
BLK = 1024
TRASH = NPAD - 2


def _mesh():
    return plsc.VectorSubcoreMesh(
        core_axis_name="c", subcore_axis_name="s", num_cores=NC, num_subcores=NS
    )


@functools.partial(
    pl.kernel,
    out_type=jax.ShapeDtypeStruct((NC, NPAD), jnp.float32),
    mesh=_mesh(),
    scratch_types=[
        pltpu.VMEM((CHUNKS, BD), jnp.int32),
        pltpu.VMEM((BD,), jnp.float32),
        pltpu.VMEM((RPT,), jnp.float32),
        pltpu.VMEM_SHARED((NPAD,), jnp.float32),
    ],
)
def _deg_call(col_hbm, z1_hbm, out_hbm, idx_v, ones_v, rbuf_v, deg_sh):
    c = lax.axis_index("c")
    s = lax.axis_index("s")
    wid = c * NS + s
    pltpu.sync_copy(z1_hbm, rbuf_v)
    pltpu.sync_copy(rbuf_v, deg_sh.at[pl.ds(s * RPT, RPT)])
    for i in range(BD // 16):
        ones_v[pl.ds(i * 16, 16)] = jnp.full((16,), 1.0, jnp.float32)
    pltpu.sync_copy(col_hbm.at[wid], idx_v)
    plsc.subcore_barrier()

    @pl.loop(0, CHUNKS)
    def _(j):
        pltpu.sync_copy(ones_v, deg_sh.at[idx_v.at[j]], add=True)

    plsc.subcore_barrier()
    pltpu.sync_copy(deg_sh.at[pl.ds(s * RPT, RPT)], rbuf_v)
    pltpu.sync_copy(rbuf_v, out_hbm.at[c, pl.ds(s * RPT, RPT)])


@functools.partial(
    pl.kernel,
    out_type=jax.ShapeDtypeStruct((NC, NPAD, D), jnp.float32),
    mesh=_mesh(),
    scratch_types=[
        pltpu.VMEM((SBC, B), jnp.int32),
        pltpu.VMEM((SBC, B), jnp.int32),
        pltpu.VMEM((B, D), jnp.float32),
        pltpu.VMEM((B, D), jnp.float32),
        pltpu.VMEM((B, D), jnp.float32),
        pltpu.VMEM((ZR, D), jnp.float32),
        pltpu.VMEM_SHARED((NPAD, D), jnp.float32),
        pltpu.SemaphoreType.DMA,
        pltpu.SemaphoreType.DMA,
        pltpu.SemaphoreType.DMA,
    ],
)
def _prop_call(t_hbm, row_hbm, col_hbm, z2_hbm, out_hbm,
               ridx_v, cidx_v, rows0, rows1, rows2, zbuf_v, acc_sh,
               sg0, sg1, sg2):
    c = lax.axis_index("c")
    s = lax.axis_index("s")
    wid = c * NS + s
    rows = (rows0, rows1, rows2)
    sg = (sg0, sg1, sg2)

    def gather(j, k):
        pltpu.async_copy(t_hbm.at[ridx_v.at[j]], rows[k], sg[k])

    def gwait(j, k):
        pltpu.make_async_copy(t_hbm.at[ridx_v.at[j]], rows[k], sg[k]).wait()

    def scat(j, k):
        pltpu.sync_copy(rows[k], acc_sh.at[cidx_v.at[j]], add=True)
    pltpu.sync_copy(z2_hbm, zbuf_v)
    for r in range(RPT // ZR):
        pltpu.sync_copy(zbuf_v, acc_sh.at[pl.ds(s * RPT + r * ZR, ZR)])
    plsc.subcore_barrier()

    @pl.loop(0, SB)
    def _(q):
        pltpu.sync_copy(row_hbm.at[wid, q], ridx_v)
        pltpu.sync_copy(col_hbm.at[wid, q], cidx_v)
        gather(0, 0)
        gather(1, 1)
        gwait(0, 0)
        gather(2, 2)
        scat(0, 0)

        @pl.loop(1, SBC - 2, step=3)
        def _(j):
            for u in range(3):
                k = (1 + u) % 3
                gwait(j + u, k)
                gather(j + u + 2, (k + 2) % 3)
                scat(j + u, k)

        gwait(SBC - 2, (SBC - 2) % 3)
        scat(SBC - 2, (SBC - 2) % 3)
        gwait(SBC - 1, (SBC - 1) % 3)
        scat(SBC - 1, (SBC - 1) % 3)

    plsc.subcore_barrier()
    for r in range(RPT // ZR):
        base = s * RPT + r * ZR
        pltpu.sync_copy(acc_sh.at[pl.ds(base, ZR)], zbuf_v)
        pltpu.sync_copy(zbuf_v, out_hbm.at[c, pl.ds(base, ZR)])


def _prep_body(degp_ref, x_ref, t0_ref, dinv_ref):
    deg = degp_ref[0] + degp_ref[1]
    di = jnp.where(deg > 0.0, lax.rsqrt(deg), 0.0)
    dinv_ref[...] = di
    t0_ref[...] = di * x_ref[...]


_prep_call = pl.pallas_call(
    _prep_body,
    grid=(NPAD // BLK,),
    in_specs=[
        pl.BlockSpec((2, BLK, 1), lambda i: (0, i, 0)),
        pl.BlockSpec((BLK, D), lambda i: (i, 0)),
    ],
    out_specs=[
        pl.BlockSpec((BLK, D), lambda i: (i, 0)),
        pl.BlockSpec((BLK, 1), lambda i: (i, 0)),
    ],
    out_shape=[
        jax.ShapeDtypeStruct((NPAD, D), jnp.float32),
        jax.ShapeDtypeStruct((NPAD, 1), jnp.float32),
    ],
)


def _comb_body(p_ref, dinv_ref, r_ref, t_ref, rout_ref):
    ssum = p_ref[0] + p_ref[1]
    di = dinv_ref[...]
    contrib = di * ssum
    t_ref[...] = di * contrib
    rout_ref[...] = r_ref[...] + contrib


_comb_call = pl.pallas_call(
    _comb_body,
    grid=(NPAD // BLK,),
    in_specs=[
        pl.BlockSpec((2, BLK, D), lambda i: (0, i, 0)),
        pl.BlockSpec((BLK, 1), lambda i: (i, 0)),
        pl.BlockSpec((BLK, D), lambda i: (i, 0)),
    ],
    out_specs=[
        pl.BlockSpec((BLK, D), lambda i: (i, 0)),
        pl.BlockSpec((BLK, D), lambda i: (i, 0)),
    ],
    out_shape=[
        jax.ShapeDtypeStruct((NPAD, D), jnp.float32),
        jax.ShapeDtypeStruct((NPAD, D), jnp.float32),
    ],
)


def _fin_body(p_ref, dinv_ref, r_ref, o_ref):
    ssum = p_ref[0] + p_ref[1]
    o_ref[...] = (r_ref[...] + dinv_ref[...] * ssum) * 0.25


_fin_call = pl.pallas_call(
    _fin_body,
    grid=(NPAD // BLK,),
    in_specs=[
        pl.BlockSpec((2, BLK, D), lambda i: (0, i, 0)),
        pl.BlockSpec((BLK, 1), lambda i: (i, 0)),
        pl.BlockSpec((BLK, D), lambda i: (i, 0)),
    ],
    out_specs=pl.BlockSpec((BLK, D), lambda i: (i, 0)),
    out_shape=jax.ShapeDtypeStruct((NPAD, D), jnp.float32),
)


def kernel(x, adj_t):
    a0 = adj_t[0].astype(jnp.int32)
    a1 = adj_t[1].astype(jnp.int32)
    row2 = jnp.concatenate([a0, a1]).reshape(NW, EPT)
    col2 = jnp.concatenate([a1, a0]).reshape(NW, EPT)
    col_deg = col2.reshape(NW, CHUNKS, BD)
    pad_r = jnp.zeros((NW, EPTP - EPT), jnp.int32)
    pad_c = jnp.full((NW, EPTP - EPT), TRASH, jnp.int32)
    row = jnp.concatenate([row2, pad_r], axis=1).reshape(NW, SB, SBC, B)
    col = jnp.concatenate([col2, pad_c], axis=1).reshape(NW, SB, SBC, B)
    xpad = jnp.concatenate([x, jnp.zeros((NPAD - N, D), jnp.float32)], axis=0)
    z1 = jnp.zeros((RPT,), jnp.float32)
    z2 = jnp.zeros((ZR, D), jnp.float32)

    degp = _deg_call(col_deg, z1)
    t, dinv = _prep_call(degp.reshape(NC, NPAD, 1), xpad)
    r = xpad
    for layer in range(3):
        p = _prop_call(t, row, col, z2)
        if layer < 2:
            t, r = _comb_call(p, dinv, r)
        else:
            out = _fin_call(p, dinv, r)
    return out[:N]


# speedup vs baseline: 3.0502x; 1.5677x over previous
import functools

import jax
import jax.numpy as jnp
from jax import lax
from jax.experimental import pallas as pl
from jax.experimental.pallas import tpu as pltpu
from jax.experimental.pallas import tpu_sc as plsc

N = 10000
D = 128
NPAD = 10240
NC = 2
NS = 16
NW = NC * NS
E2 = 640000
EPT = E2 // NW
BD = 80
CHUNKS = EPT // BD
B = 80
SBC = 50
SB = 5
NB = SB * SBC
EPTP = NB * B
RPT = NPAD // NS
ZR = 64
BLK = 1024
TRASH = NPAD - 2


def _mesh():
    return plsc.VectorSubcoreMesh(
        core_axis_name="c", subcore_axis_name="s", num_cores=NC, num_subcores=NS
    )


@functools.partial(
    pl.kernel,
    out_type=jax.ShapeDtypeStruct((NC, NPAD), jnp.float32),
    mesh=_mesh(),
    scratch_types=[
        pltpu.VMEM((CHUNKS, BD), jnp.int32),
        pltpu.VMEM((BD,), jnp.float32),
        pltpu.VMEM_SHARED((NPAD,), jnp.float32),
    ],
)
def _deg_call(col_hbm, z1_hbm, out_hbm, idx_v, ones_v, deg_sh):
    c = lax.axis_index("c")
    s = lax.axis_index("s")
    wid = c * NS + s
    pltpu.sync_copy(z1_hbm, deg_sh.at[pl.ds(s * RPT, RPT)])
    for i in range(BD // 16):
        ones_v[pl.ds(i * 16, 16)] = jnp.full((16,), 1.0, jnp.float32)
    pltpu.sync_copy(col_hbm.at[wid], idx_v)
    plsc.subcore_barrier()

    @pl.loop(0, CHUNKS)
    def _(j):
        pltpu.sync_copy(ones_v, deg_sh.at[idx_v.at[j]], add=True)

    plsc.subcore_barrier()
    pltpu.sync_copy(deg_sh.at[pl.ds(s * RPT, RPT)],
                    out_hbm.at[c, pl.ds(s * RPT, RPT)])


@functools.partial(
    pl.kernel,
    out_type=jax.ShapeDtypeStruct((NC, NPAD, D), jnp.float32),
    mesh=_mesh(),
    scratch_types=[
        pltpu.VMEM((SBC, B), jnp.int32),
        pltpu.VMEM((SBC, B), jnp.int32),
        pltpu.VMEM((B, D), jnp.float32),
        pltpu.VMEM((B, D), jnp.float32),
        pltpu.VMEM_SHARED((NPAD, D), jnp.float32),
        pltpu.SemaphoreType.DMA,
        pltpu.SemaphoreType.DMA,
    ],
)
def _prop_call(t_hbm, row_hbm, col_hbm, z2_hbm, out_hbm,
               ridx_v, cidx_v, rows_a, rows_b, acc_sh, sem_a, sem_b):
    c = lax.axis_index("c")
    s = lax.axis_index("s")
    wid = c * NS + s
    rows = (rows_a, rows_b)
    sg = (sem_a, sem_b)

    def gather(j, k):
        pltpu.async_copy(t_hbm.at[ridx_v.at[j]], rows[k], sg[k])

    def gwait(j, k):
        pltpu.make_async_copy(t_hbm.at[ridx_v.at[j]], rows[k], sg[k]).wait()

    def scat(j, k):
        pltpu.sync_copy(rows[k], acc_sh.at[cidx_v.at[j]], add=True)
    pltpu.sync_copy(z2_hbm, acc_sh.at[pl.ds(s * RPT, RPT)])
    plsc.subcore_barrier()

    @pl.loop(0, SB)
    def _(q):
        pltpu.sync_copy(row_hbm.at[wid, q], ridx_v)
        pltpu.sync_copy(col_hbm.at[wid, q], cidx_v)
        gather(0, 0)

        @pl.loop(0, SBC // 2 - 1)
        def _(p):
            j = p * 2
            gather(j + 1, 1)
            gwait(j, 0)
            scat(j, 0)
            gather(j + 2, 0)
            gwait(j + 1, 1)
            scat(j + 1, 1)

        gather(SBC - 1, 1)
        gwait(SBC - 2, 0)
        scat(SBC - 2, 0)
        gwait(SBC - 1, 1)
        scat(SBC - 1, 1)

    plsc.subcore_barrier()
    pltpu.sync_copy(acc_sh.at[pl.ds(s * RPT, RPT)],
                    out_hbm.at[c, pl.ds(s * RPT, RPT)])


def _prep_body(degp_ref, x_ref, t0_ref, dinv_ref):
    deg = degp_ref[0] + degp_ref[1]
    di = jnp.where(deg > 0.0, lax.rsqrt(deg), 0.0)
    dinv_ref[...] = di
    t0_ref[...] = di * x_ref[...]


_prep_call = pl.pallas_call(
    _prep_body,
    grid=(NPAD // BLK,),
    in_specs=[
        pl.BlockSpec((2, BLK, 1), lambda i: (0, i, 0)),
        pl.BlockSpec((BLK, D), lambda i: (i, 0)),
    ],
    out_specs=[
        pl.BlockSpec((BLK, D), lambda i: (i, 0)),
        pl.BlockSpec((BLK, 1), lambda i: (i, 0)),
    ],
    out_shape=[
        jax.ShapeDtypeStruct((NPAD, D), jnp.float32),
        jax.ShapeDtypeStruct((NPAD, 1), jnp.float32),
    ],
)


def _comb_body(p_ref, dinv_ref, r_ref, t_ref, rout_ref):
    ssum = p_ref[0] + p_ref[1]
    di = dinv_ref[...]
    contrib = di * ssum
    t_ref[...] = di * contrib
    rout_ref[...] = r_ref[...] + contrib


_comb_call = pl.pallas_call(
    _comb_body,
    grid=(NPAD // BLK,),
    in_specs=[
        pl.BlockSpec((2, BLK, D), lambda i: (0, i, 0)),
        pl.BlockSpec((BLK, 1), lambda i: (i, 0)),
        pl.BlockSpec((BLK, D), lambda i: (i, 0)),
    ],
    out_specs=[
        pl.BlockSpec((BLK, D), lambda i: (i, 0)),
        pl.BlockSpec((BLK, D), lambda i: (i, 0)),
    ],
    out_shape=[
        jax.ShapeDtypeStruct((NPAD, D), jnp.float32),
        jax.ShapeDtypeStruct((NPAD, D), jnp.float32),
    ],
)


def _fin_body(p_ref, dinv_ref, r_ref, o_ref):
    ssum = p_ref[0] + p_ref[1]
    o_ref[...] = (r_ref[...] + dinv_ref[...] * ssum) * 0.25


_fin_call = pl.pallas_call(
    _fin_body,
    grid=(NPAD // BLK,),
    in_specs=[
        pl.BlockSpec((2, BLK, D), lambda i: (0, i, 0)),
        pl.BlockSpec((BLK, 1), lambda i: (i, 0)),
        pl.BlockSpec((BLK, D), lambda i: (i, 0)),
    ],
    out_specs=pl.BlockSpec((BLK, D), lambda i: (i, 0)),
    out_shape=jax.ShapeDtypeStruct((NPAD, D), jnp.float32),
)


def kernel(x, adj_t):
    a0 = adj_t[0].astype(jnp.int32)
    a1 = adj_t[1].astype(jnp.int32)
    row = jnp.concatenate([a0, a1]).reshape(NW, SB, SBC, B)
    col = jnp.concatenate([a1, a0]).reshape(NW, SB, SBC, B)
    col_deg = col.reshape(NW, CHUNKS, BD)
    xpad = jnp.concatenate([x, jnp.zeros((NPAD - N, D), jnp.float32)], axis=0)
    z1 = jnp.zeros((RPT,), jnp.float32)
    z2 = jnp.zeros((RPT, D), jnp.float32)

    degp = _deg_call(col_deg, z1)
    t, dinv = _prep_call(degp.reshape(NC, NPAD, 1), xpad)
    r = xpad
    for layer in range(3):
        p = _prop_call(t, row, col, z2)
        if layer < 2:
            t, r = _comb_call(p, dinv, r)
        else:
            out = _fin_call(p, dinv, r)
    return out[:N]


# speedup vs baseline: 3.0719x; 1.0071x over previous
import functools

import jax
import jax.numpy as jnp
from jax import lax
from jax.experimental import pallas as pl
from jax.experimental.pallas import tpu as pltpu
from jax.experimental.pallas import tpu_sc as plsc

N = 10000
D = 128
NPAD = 10240
NC = 2
NS = 16
NW = NC * NS
E = 320000
EPT = 2 * E // NW
BD = 80
CHUNKS = EPT // BD
HCH = CHUNKS // 2
B = 80
SBC = 50
SB = 5
RPT = NPAD // NS
RPTN = N // NS
BLK = 1000


def _mesh():
    return plsc.VectorSubcoreMesh(
        core_axis_name="c", subcore_axis_name="s", num_cores=NC, num_subcores=NS
    )


@functools.partial(
    pl.kernel,
    out_type=jax.ShapeDtypeStruct((NC, NPAD), jnp.float32),
    mesh=_mesh(),
    scratch_types=[
        pltpu.VMEM((CHUNKS, BD), jnp.int32),
        pltpu.VMEM((BD,), jnp.float32),
        pltpu.VMEM_SHARED((NPAD,), jnp.float32),
    ],
)
def _deg_call(a0_hbm, a1_hbm, z1_hbm, out_hbm, idx_v, ones_v, deg_sh):
    c = lax.axis_index("c")
    s = lax.axis_index("s")
    wid = c * NS + s
    pltpu.sync_copy(z1_hbm, deg_sh.at[pl.ds(s * RPT, RPT)])
    for i in range(BD // 16):
        ones_v[pl.ds(i * 16, 16)] = jnp.full((16,), 1.0, jnp.float32)
    pltpu.sync_copy(a0_hbm.at[wid], idx_v.at[pl.ds(0, HCH)])
    pltpu.sync_copy(a1_hbm.at[wid], idx_v.at[pl.ds(HCH, HCH)])
    plsc.subcore_barrier()

    @pl.loop(0, CHUNKS)
    def _(j):
        pltpu.sync_copy(ones_v, deg_sh.at[idx_v.at[j]], add=True)

    plsc.subcore_barrier()
    pltpu.sync_copy(deg_sh.at[pl.ds(s * RPT, RPT)],
                    out_hbm.at[c, pl.ds(s * RPT, RPT)])


@functools.partial(
    pl.kernel,
    out_type=jax.ShapeDtypeStruct((NC, NPAD, D), jnp.float32),
    mesh=_mesh(),
    scratch_types=[
        pltpu.VMEM((SBC, B), jnp.int32),
        pltpu.VMEM((SBC, B), jnp.int32),
        pltpu.VMEM((B, D), jnp.float32),
        pltpu.VMEM((B, D), jnp.float32),
        pltpu.VMEM_SHARED((NPAD, D), jnp.float32),
        pltpu.SemaphoreType.DMA,
        pltpu.SemaphoreType.DMA,
    ],
)
def _prop_call(t_hbm, a0_hbm, a1_hbm, z2_hbm, out_hbm,
               ridx_v, cidx_v, rows_a, rows_b, acc_sh, sem_a, sem_b):
    c = lax.axis_index("c")
    s = lax.axis_index("s")
    rows = (rows_a, rows_b)
    sg = (sem_a, sem_b)

    def gather(j, k):
        pltpu.async_copy(t_hbm.at[ridx_v.at[j]], rows[k], sg[k])

    def gwait(j, k):
        pltpu.make_async_copy(t_hbm.at[ridx_v.at[j]], rows[k], sg[k]).wait()

    def scat(j, k):
        pltpu.sync_copy(rows[k], acc_sh.at[cidx_v.at[j]], add=True)

    pltpu.sync_copy(z2_hbm, acc_sh.at[pl.ds(s * RPT, RPT)])
    plsc.subcore_barrier()

    @pl.loop(0, SB)
    def _(q):
        @pl.when(c == 0)
        def _():
            pltpu.sync_copy(a0_hbm.at[s, q], ridx_v)
            pltpu.sync_copy(a1_hbm.at[s, q], cidx_v)

        @pl.when(c == 1)
        def _():
            pltpu.sync_copy(a1_hbm.at[s, q], ridx_v)
            pltpu.sync_copy(a0_hbm.at[s, q], cidx_v)

        gather(0, 0)

        @pl.loop(0, SBC // 2 - 1)
        def _(p):
            j = p * 2
            gather(j + 1, 1)
            gwait(j, 0)
            scat(j, 0)
            gather(j + 2, 0)
            gwait(j + 1, 1)
            scat(j + 1, 1)

        gather(SBC - 1, 1)
        gwait(SBC - 2, 0)
        scat(SBC - 2, 0)
        gwait(SBC - 1, 1)
        scat(SBC - 1, 1)

    plsc.subcore_barrier()
    pltpu.sync_copy(acc_sh.at[pl.ds(s * RPT, RPT)],
                    out_hbm.at[c, pl.ds(s * RPT, RPT)])


def _prep_body(degp_ref, x_ref, t0_ref, dinv_ref):
    deg = degp_ref[0] + degp_ref[1]
    di = jnp.where(deg > 0.0, lax.rsqrt(deg), 0.0)
    dinv_ref[...] = di
    t0_ref[...] = di * x_ref[...]


_prep_call = pl.pallas_call(
    _prep_body,
    grid=(N // BLK,),
    in_specs=[
        pl.BlockSpec((2, BLK, 1), lambda i: (0, i, 0)),
        pl.BlockSpec((BLK, D), lambda i: (i, 0)),
    ],
    out_specs=[
        pl.BlockSpec((BLK, D), lambda i: (i, 0)),
        pl.BlockSpec((BLK, 1), lambda i: (i, 0)),
    ],
    out_shape=[
        jax.ShapeDtypeStruct((N, D), jnp.float32),
        jax.ShapeDtypeStruct((N, 1), jnp.float32),
    ],
)


def _comb_body(p_ref, dinv_ref, r_ref, t_ref, rout_ref):
    ssum = p_ref[0] + p_ref[1]
    di = dinv_ref[...]
    contrib = di * ssum
    t_ref[...] = di * contrib
    rout_ref[...] = r_ref[...] + contrib


_comb_call = pl.pallas_call(
    _comb_body,
    grid=(N // BLK,),
    in_specs=[
        pl.BlockSpec((2, BLK, D), lambda i: (0, i, 0)),
        pl.BlockSpec((BLK, 1), lambda i: (i, 0)),
        pl.BlockSpec((BLK, D), lambda i: (i, 0)),
    ],
    out_specs=[
        pl.BlockSpec((BLK, D), lambda i: (i, 0)),
        pl.BlockSpec((BLK, D), lambda i: (i, 0)),
    ],
    out_shape=[
        jax.ShapeDtypeStruct((N, D), jnp.float32),
        jax.ShapeDtypeStruct((N, D), jnp.float32),
    ],
)


def _fin_body(p_ref, dinv_ref, r_ref, o_ref):
    ssum = p_ref[0] + p_ref[1]
    o_ref[...] = (r_ref[...] + dinv_ref[...] * ssum) * 0.25


_fin_call = pl.pallas_call(
    _fin_body,
    grid=(N // BLK,),
    in_specs=[
        pl.BlockSpec((2, BLK, D), lambda i: (0, i, 0)),
        pl.BlockSpec((BLK, 1), lambda i: (i, 0)),
        pl.BlockSpec((BLK, D), lambda i: (i, 0)),
    ],
    out_specs=pl.BlockSpec((BLK, D), lambda i: (i, 0)),
    out_shape=jax.ShapeDtypeStruct((N, D), jnp.float32),
)


def kernel(x, adj_t):
    a0 = adj_t[0].astype(jnp.int32)
    a1 = adj_t[1].astype(jnp.int32)
    a0d = a0.reshape(NW, HCH, BD)
    a1d = a1.reshape(NW, HCH, BD)
    a0p = a0.reshape(NS, SB, SBC, B)
    a1p = a1.reshape(NS, SB, SBC, B)
    z1 = jnp.zeros((RPT,), jnp.float32)
    z2 = jnp.zeros((RPT, D), jnp.float32)

    degp = _deg_call(a0d, a1d, z1)
    t, dinv = _prep_call(degp.reshape(NC, NPAD, 1), x)
    r = x
    for layer in range(3):
        p = _prop_call(t, a0p, a1p, z2)
        if layer < 2:
            t, r = _comb_call(p, dinv, r)
        else:
            out = _fin_call(p, dinv, r)
    return out
